# baseline (speedup 1.0000x reference)
import functools

import numpy as np
import jax
import jax.numpy as jnp
from jax import lax
from jax.experimental import pallas as pl
from jax.experimental.pallas import tpu as pltpu
from jax.experimental.pallas import tpu_sc as plsc

OUT_H = 7
OUT_W = 7
NBIN = OUT_H * OUT_W
SPATIAL_SCALE = 0.125
L = 16
NC, NS = 2, 16
NW = NC * NS


def _sc_roi_align(feat_rows, roif, H, W, C, N):
    nbins = N * NBIN
    bins_per_w = nbins // NW
    G = 8
    nbatch = bins_per_w // G
    rois_per_w = N // NW
    fH = float(H)
    fW = float(W)

    mesh = plsc.VectorSubcoreMesh(
        core_axis_name="c", subcore_axis_name="s",
        num_cores=NC, num_subcores=NS)

    @functools.partial(
        pl.kernel,
        out_type=jax.ShapeDtypeStruct((nbins, 1, C), jnp.bfloat16),
        mesh=mesh,
        compiler_params=pltpu.CompilerParams(
            needs_layout_passes=False, use_tc_tiling_on_sc=False),
        scratch_types=[
            pltpu.VMEM((rois_per_w, L), jnp.float32),
            pltpu.VMEM((G * L,), jnp.int32),
            pltpu.VMEM((G * L,), jnp.int32),
            pltpu.VMEM((G * L,), jnp.float32),
            pltpu.VMEM((G * L,), jnp.float32),
            pltpu.VMEM((G * L, C), jnp.bfloat16),
            pltpu.VMEM((G * L, C), jnp.bfloat16),
            pltpu.VMEM((G, 1, C), jnp.bfloat16),
            pltpu.VMEM((G, 1, C), jnp.bfloat16),
            pltpu.SemaphoreType.DMA,
            pltpu.SemaphoreType.DMA,
            pltpu.SemaphoreType.DMA,
            pltpu.SemaphoreType.DMA,
        ],
    )
    def k(feat_hbm, roif_hbm, out_hbm, roi_v, idx_a, idx_b, w_a, w_b,
          rows_a, rows_b, outb_a, outb_b, gsem0, gsem1, osem0, osem1):
        idxs = (idx_a, idx_b)
        ws = (w_a, w_b)
        rows = (rows_a, rows_b)
        outbs = (outb_a, outb_b)
        gsems = (gsem0, gsem1)
        osems = (osem0, osem1)

        wid = lax.axis_index("s") * NC + lax.axis_index("c")
        roi0 = wid * rois_per_w
        bin0 = wid * bins_per_w
        pltpu.sync_copy(roif_hbm.at[pl.ds(roi0, rois_per_w)], roi_v)

        lanes = lax.iota(jnp.int32, L)
        sample = lanes >> 2
        corner = lanes & 3
        iy_l = 0.25 + 0.5 * (sample >> 1).astype(jnp.float32)
        ix_l = 0.25 + 0.5 * (sample & 1).astype(jnp.float32)
        dyi = corner >> 1
        dxi = corner & 1
        dy0 = dyi == 0
        dx0 = dxi == 0

        def prep(bt, s):
            def prep_body(b, c2):
                lb = bt * G + b
                n_loc = lb // NBIN
                r = lb - n_loc * NBIN
                ph = r // OUT_W
                pw = r - ph * OUT_W
                rv = roi_v[n_loc, :]
                cxs = rv[0]
                cys = rv[1]
                bws = rv[2]
                bhs = rv[3]
                css = rv[4]
                sns = rv[5]
                basi = rv[6].astype(jnp.int32)
                phf = ph.astype(jnp.float32)
                pwf = pw.astype(jnp.float32)
                yy = bhs * (phf + (iy_l - 3.5))
                xx = bws * (pwf + (ix_l - 3.5))
                y = yy * css - xx * sns + cys
                x = yy * sns + xx * css + cxs
                ok = (y > -1.0) & (y < fH) & (x > -1.0) & (x < fW)
                vf = jnp.where(ok, 0.25, 0.0)
                ycl = jnp.clip(y, 0.0, fH - 1.0)
                xcl = jnp.clip(x, 0.0, fW - 1.0)
                y0 = jnp.minimum(ycl.astype(jnp.int32), H - 2)
                x0 = jnp.minimum(xcl.astype(jnp.int32), W - 2)
                ly = ycl - y0.astype(jnp.float32)
                lx = xcl - x0.astype(jnp.float32)
                wgt = jnp.where(dy0, 1.0 - ly, ly) * jnp.where(dx0, 1.0 - lx, lx) * vf
                idx = basi + (y0 + dyi) * W + (x0 + dxi)
                idxs[s][pl.ds(b * L, L)] = idx
                ws[s][pl.ds(b * L, L)] = wgt
                return c2

            lax.fori_loop(0, G, prep_body, 0, unroll=False)

        def gather_copy(s):
            return pltpu.make_async_copy(
                feat_hbm.at[idxs[s]], rows[s], gsems[s])

        def out_copy(bt, s):
            return pltpu.make_async_copy(
                outbs[s], out_hbm.at[pl.ds(bin0 + bt * G, G)], osems[s])

        def fma(s):
            rows_s = rows[s]
            outb_s = outbs[s]
            w_s = ws[s]

            def fma_body(b, c2):
                b16 = b * L
                wv = w_s[pl.ds(b16, L)]
                wbf = []
                for t in range(L):
                    wsp = jnp.full((L,), wv[t], jnp.float32)
                    wbf.append(plsc.pack(
                        wsp, wsp, format=plsc.PackFormat.INTERLEAVED))
                for q in range(C // (2 * L)):
                    sl = pl.ds(q * 2 * L, 2 * L)
                    acc = wbf[0] * rows_s[b16, sl]
                    for t in range(1, L):
                        acc = acc + wbf[t] * rows_s[b16 + t, sl]
                    outb_s[b, 0, sl] = acc
                return c2

            lax.fori_loop(0, G, fma_body, 0, unroll=False)

        prep(0, 0)
        gather_copy(0).start()

        def pair_body(p, carry):
            for s in (0, 1):
                bt = 2 * p + s
                o = 1 - s

                @pl.when(bt < nbatch - 1)
                def _():
                    prep(bt + 1, o)
                    gather_copy(o).start()

                gather_copy(s).wait()

                @pl.when(bt >= 2)
                def _():
                    out_copy(bt - 2, s).wait()

                fma(s)
                out_copy(bt, s).start()
            return carry

        lax.fori_loop(0, nbatch // 2, pair_body, 0, unroll=False)
        out_copy(nbatch - 2, 0).wait()
        out_copy(nbatch - 1, 1).wait()

    return k(feat_rows, roif)


def kernel(features, rois):
    B, C, H, W = features.shape
    N = rois.shape[0]
    feat_rows = (jnp.transpose(features, (0, 2, 3, 1))
                 .astype(jnp.bfloat16).reshape(B * H * W, C))
    offset = 0.5
    cx = rois[:, 1] * SPATIAL_SCALE - offset
    cy = rois[:, 2] * SPATIAL_SCALE - offset
    bw = rois[:, 3] * (SPATIAL_SCALE / OUT_W)
    bh = rois[:, 4] * (SPATIAL_SCALE / OUT_H)
    theta = rois[:, 5]
    base = rois[:, 0].astype(jnp.int32).astype(jnp.float32) * float(H * W)
    z = jnp.zeros_like(cx)
    roif = jnp.stack(
        [cx, cy, bw, bh, jnp.cos(theta), jnp.sin(theta), base,
         z, z, z, z, z, z, z, z, z], axis=1)
    out = _sc_roi_align(feat_rows, roif, H, W, C, N)
    return (out.astype(jnp.float32)
            .reshape(N, OUT_H, OUT_W, C).transpose(0, 3, 1, 2))


# speedup vs baseline: 1.3645x; 1.3645x over previous
import functools

import jax
import jax.numpy as jnp
from jax import lax
from jax.experimental import pallas as pl
from jax.experimental.pallas import tpu as pltpu
from jax.experimental.pallas import tpu_sc as plsc

OUT_H = 7
OUT_W = 7
NBIN = OUT_H * OUT_W
SPATIAL_SCALE = 0.125
L = 16
NC, NS = 2, 16
NW = NC * NS


def _sc_roi_align(feat_rows, roif, H, W, C, N):
    nbins = N * NBIN
    bins_per_w = nbins // NW
    G = OUT_W
    nbatch = bins_per_w // G
    rois_per_w = N // NW
    RSZ = C * NBIN
    fH = float(H)
    fW = float(W)

    mesh = plsc.VectorSubcoreMesh(
        core_axis_name="c", subcore_axis_name="s",
        num_cores=NC, num_subcores=NS)

    @functools.partial(
        pl.kernel,
        out_type=jax.ShapeDtypeStruct((N, RSZ), jnp.float32),
        mesh=mesh,
        compiler_params=pltpu.CompilerParams(
            needs_layout_passes=False, use_tc_tiling_on_sc=False),
        scratch_types=[
            pltpu.VMEM((rois_per_w, L), jnp.float32),
            pltpu.VMEM((G * L,), jnp.int32),
            pltpu.VMEM((G * L,), jnp.int32),
            pltpu.VMEM((G * L,), jnp.float32),
            pltpu.VMEM((G * L,), jnp.float32),
            pltpu.VMEM((G * L, C), jnp.bfloat16),
            pltpu.VMEM((G * L, C), jnp.bfloat16),
            pltpu.VMEM((2 * RSZ,), jnp.float32),
            pltpu.SemaphoreType.DMA,
            pltpu.SemaphoreType.DMA,
            pltpu.SemaphoreType.DMA,
            pltpu.SemaphoreType.DMA,
        ],
    )
    def k(feat_hbm, roif_hbm, out_hbm, roi_v, idx_a, idx_b, w_a, w_b,
          rows_a, rows_b, stage_v, gsem0, gsem1, osem0, osem1):
        idxs = (idx_a, idx_b)
        ws = (w_a, w_b)
        rows = (rows_a, rows_b)
        gsems = (gsem0, gsem1)
        osems = (osem0, osem1)

        wid = lax.axis_index("s") * NC + lax.axis_index("c")
        roi0 = wid * rois_per_w
        pltpu.sync_copy(roif_hbm.at[pl.ds(roi0, rois_per_w)], roi_v)

        lanes = lax.iota(jnp.int32, L)
        sample = lanes >> 2
        corner = lanes & 3
        iy_l = 0.25 + 0.5 * (sample >> 1).astype(jnp.float32)
        ix_l = 0.25 + 0.5 * (sample & 1).astype(jnp.float32)
        dyi = corner >> 1
        dxi = corner & 1
        dy0 = dyi == 0
        dx0 = dxi == 0
        lane98 = lanes * (2 * NBIN)

        def prep(bt, s):
            def prep_body(b, c2):
                lb = bt * G + b
                n_loc = lb // NBIN
                r = lb - n_loc * NBIN
                ph = r // OUT_W
                pw = r - ph * OUT_W
                rv = roi_v[n_loc, :]
                cxs = rv[0]
                cys = rv[1]
                bws = rv[2]
                bhs = rv[3]
                css = rv[4]
                sns = rv[5]
                basi = rv[6].astype(jnp.int32)
                phf = ph.astype(jnp.float32)
                pwf = pw.astype(jnp.float32)
                yy = bhs * (phf + (iy_l - 3.5))
                xx = bws * (pwf + (ix_l - 3.5))
                y = yy * css - xx * sns + cys
                x = yy * sns + xx * css + cxs
                ok = (y > -1.0) & (y < fH) & (x > -1.0) & (x < fW)
                vf = jnp.where(ok, 0.25, 0.0)
                ycl = jnp.clip(y, 0.0, fH - 1.0)
                xcl = jnp.clip(x, 0.0, fW - 1.0)
                y0 = jnp.minimum(ycl.astype(jnp.int32), H - 2)
                x0 = jnp.minimum(xcl.astype(jnp.int32), W - 2)
                ly = ycl - y0.astype(jnp.float32)
                lx = xcl - x0.astype(jnp.float32)
                wgt = jnp.where(dy0, 1.0 - ly, ly) * jnp.where(dx0, 1.0 - lx, lx) * vf
                idx = basi + (y0 + dyi) * W + (x0 + dxi)
                idxs[s][pl.ds(b * L, L)] = idx
                ws[s][pl.ds(b * L, L)] = wgt
                return c2

            lax.fori_loop(0, G, prep_body, 0, unroll=False)

        def gather_copy(s):
            return pltpu.make_async_copy(
                feat_hbm.at[idxs[s]], rows[s], gsems[s])

        def out_copy(n_loc, sem):
            return pltpu.make_async_copy(
                stage_v.at[pl.ds((n_loc % 2) * RSZ, RSZ)],
                out_hbm.at[roi0 + n_loc], sem)

        def fma(bt, s):
            rows_s = rows[s]
            w_s = ws[s]

            def fma_body(b, c2):
                lb = bt * G + b
                n_loc = lb // NBIN
                bin_r = lb - n_loc * NBIN
                sbase = (n_loc % 2) * RSZ + bin_r
                b16 = b * L
                wv = w_s[pl.ds(b16, L)]
                wbf = []
                for t in range(L):
                    wsp = jnp.full((L,), wv[t], jnp.float32)
                    wbf.append(plsc.pack(
                        wsp, wsp, format=plsc.PackFormat.INTERLEAVED))
                for q in range(C // (2 * L)):
                    sl = pl.ds(q * 2 * L, 2 * L)
                    acc = wbf[0] * rows_s[b16, sl]
                    for t in range(1, L):
                        acc = acc + wbf[t] * rows_s[b16 + t, sl]
                    al, au = plsc.unpack(
                        acc, format=plsc.PackFormat.INTERLEAVED,
                        preferred_element_type=jnp.float32)
                    idx_e = lane98 + (sbase + q * 2 * L * NBIN)
                    plsc.store_scatter(stage_v, [idx_e], al)
                    plsc.store_scatter(stage_v, [idx_e + NBIN], au)
                return c2

            lax.fori_loop(0, G, fma_body, 0, unroll=False)

        prep(0, 0)
        gather_copy(0).start()

        def pair_body(p, carry):
            for s in (0, 1):
                bt = 2 * p + s
                o = 1 - s
                n_loc = bt // OUT_H
                j7 = bt - n_loc * OUT_H

                @pl.when(bt < nbatch - 1)
                def _():
                    prep(bt + 1, o)
                    gather_copy(o).start()

                gather_copy(s).wait()

                @pl.when((j7 == 0) & (n_loc >= 2) & (n_loc % 2 == 0))
                def _():
                    out_copy(n_loc - 2, osem0).wait()

                @pl.when((j7 == 0) & (n_loc >= 2) & (n_loc % 2 == 1))
                def _():
                    out_copy(n_loc - 2, osem1).wait()

                fma(bt, s)

                @pl.when((j7 == OUT_H - 1) & (n_loc % 2 == 0))
                def _():
                    out_copy(n_loc, osem0).start()

                @pl.when((j7 == OUT_H - 1) & (n_loc % 2 == 1))
                def _():
                    out_copy(n_loc, osem1).start()
            return carry

        lax.fori_loop(0, nbatch // 2, pair_body, 0, unroll=False)
        out_copy(rois_per_w - 2, osem0).wait()
        out_copy(rois_per_w - 1, osem1).wait()

    return k(feat_rows, roif)


def kernel(features, rois):
    B, C, H, W = features.shape
    N = rois.shape[0]
    feat_rows = (jnp.transpose(features, (0, 2, 3, 1))
                 .astype(jnp.bfloat16).reshape(B * H * W, C))
    offset = 0.5
    cx = rois[:, 1] * SPATIAL_SCALE - offset
    cy = rois[:, 2] * SPATIAL_SCALE - offset
    bw = rois[:, 3] * (SPATIAL_SCALE / OUT_W)
    bh = rois[:, 4] * (SPATIAL_SCALE / OUT_H)
    theta = rois[:, 5]
    base = rois[:, 0].astype(jnp.int32).astype(jnp.float32) * float(H * W)
    z = jnp.zeros_like(cx)
    roif = jnp.stack(
        [cx, cy, bw, bh, jnp.cos(theta), jnp.sin(theta), base,
         z, z, z, z, z, z, z, z, z], axis=1)
    out = _sc_roi_align(feat_rows, roif, H, W, C, N)
    return out.reshape(N, C, OUT_H, OUT_W)


# speedup vs baseline: 1.4554x; 1.0666x over previous
import functools

import jax
import jax.numpy as jnp
from jax import lax
from jax.experimental import pallas as pl
from jax.experimental.pallas import tpu as pltpu
from jax.experimental.pallas import tpu_sc as plsc

OUT_H = 7
OUT_W = 7
NBIN = OUT_H * OUT_W
SPATIAL_SCALE = 0.125
L = 16
NC, NS = 2, 16
NW = NC * NS


def _sc_roi_align(feat_rows, roif, H, W, C, N):
    nbins = N * NBIN
    bins_per_w = nbins // NW
    G = OUT_W
    nbatch = bins_per_w // G
    rois_per_w = N // NW
    RSZ = C * NBIN
    fH = float(H)
    fW = float(W)

    mesh = plsc.VectorSubcoreMesh(
        core_axis_name="c", subcore_axis_name="s",
        num_cores=NC, num_subcores=NS)

    @functools.partial(
        pl.kernel,
        out_type=jax.ShapeDtypeStruct((N, RSZ), jnp.float32),
        mesh=mesh,
        compiler_params=pltpu.CompilerParams(
            needs_layout_passes=False, use_tc_tiling_on_sc=False),
        scratch_types=[
            pltpu.VMEM((rois_per_w, L), jnp.float32),
            pltpu.VMEM((G * L,), jnp.int32),
            pltpu.VMEM((G * L,), jnp.int32),
            pltpu.VMEM((G * L,), jnp.float32),
            pltpu.VMEM((G * L,), jnp.float32),
            pltpu.VMEM((G * L, C), jnp.bfloat16),
            pltpu.VMEM((G * L, C), jnp.bfloat16),
            pltpu.VMEM((2 * RSZ,), jnp.float32),
            pltpu.SemaphoreType.DMA,
            pltpu.SemaphoreType.DMA,
            pltpu.SemaphoreType.DMA,
            pltpu.SemaphoreType.DMA,
        ],
    )
    def k(feat_hbm, roif_hbm, out_hbm, roi_v, idx_a, idx_b, w_a, w_b,
          rows_a, rows_b, stage_v, gsem0, gsem1, osem0, osem1):
        idxs = (idx_a, idx_b)
        ws = (w_a, w_b)
        rows = (rows_a, rows_b)
        gsems = (gsem0, gsem1)
        osems = (osem0, osem1)

        wid = lax.axis_index("s") * NC + lax.axis_index("c")
        roi0 = wid * rois_per_w
        pltpu.sync_copy(roif_hbm.at[pl.ds(roi0, rois_per_w)], roi_v)

        lanes = lax.iota(jnp.int32, L)
        sample = lanes >> 2
        corner = lanes & 3
        iy_l = 0.25 + 0.5 * (sample >> 1).astype(jnp.float32)
        ix_l = 0.25 + 0.5 * (sample & 1).astype(jnp.float32)
        dyi = corner >> 1
        dxi = corner & 1
        dy0 = dyi == 0
        dx0 = dxi == 0
        lane98 = lanes * (2 * NBIN)

        def prep(n_loc, j7, s):
            rv = roi_v[n_loc, :]

            def prep_body(b, c2):
                ph = j7
                pw = b
                cxs = rv[0]
                cys = rv[1]
                bws = rv[2]
                bhs = rv[3]
                css = rv[4]
                sns = rv[5]
                basi = rv[6].astype(jnp.int32)
                phf = ph.astype(jnp.float32)
                pwf = pw.astype(jnp.float32)
                yy = bhs * (phf + (iy_l - 3.5))
                xx = bws * (pwf + (ix_l - 3.5))
                y = yy * css - xx * sns + cys
                x = yy * sns + xx * css + cxs
                ok = (y > -1.0) & (y < fH) & (x > -1.0) & (x < fW)
                vf = jnp.where(ok, 0.25, 0.0)
                ycl = jnp.clip(y, 0.0, fH - 1.0)
                xcl = jnp.clip(x, 0.0, fW - 1.0)
                y0 = jnp.minimum(ycl.astype(jnp.int32), H - 2)
                x0 = jnp.minimum(xcl.astype(jnp.int32), W - 2)
                ly = ycl - y0.astype(jnp.float32)
                lx = xcl - x0.astype(jnp.float32)
                wgt = jnp.where(dy0, 1.0 - ly, ly) * jnp.where(dx0, 1.0 - lx, lx) * vf
                idx = basi + (y0 + dyi) * W + (x0 + dxi)
                idxs[s][pl.ds(b * L, L)] = idx
                ws[s][pl.ds(b * L, L)] = wgt
                return c2

            lax.fori_loop(0, G, prep_body, 0, unroll=False)

        def gather_copy(s):
            return pltpu.make_async_copy(
                feat_hbm.at[idxs[s]], rows[s], gsems[s])

        def out_copy(n_loc, sem):
            return pltpu.make_async_copy(
                stage_v.at[pl.ds((n_loc % 2) * RSZ, RSZ)],
                out_hbm.at[roi0 + n_loc], sem)

        def fma(n_loc, j7, s):
            rows_s = rows[s]
            w_s = ws[s]
            sbase0 = (n_loc % 2) * RSZ + j7 * OUT_W

            def fma_body(b, c2):
                sbase = sbase0 + b
                b16 = b * L
                wv = w_s[pl.ds(b16, L)]
                wbf = []
                for t in range(L):
                    wsp = jnp.full((L,), wv[t], jnp.float32)
                    wbf.append(plsc.pack(
                        wsp, wsp, format=plsc.PackFormat.INTERLEAVED))
                for q in range(C // (2 * L)):
                    sl = pl.ds(q * 2 * L, 2 * L)
                    prods = [wbf[t] * rows_s[b16 + t, sl] for t in range(L)]
                    while len(prods) > 1:
                        prods = [prods[i] + prods[i + 1]
                                 for i in range(0, len(prods), 2)]
                    acc = prods[0]
                    al, au = plsc.unpack(
                        acc, format=plsc.PackFormat.INTERLEAVED,
                        preferred_element_type=jnp.float32)
                    idx_e = lane98 + (sbase + q * 2 * L * NBIN)
                    plsc.store_scatter(stage_v, [idx_e], al)
                    plsc.store_scatter(stage_v, [idx_e + NBIN], au)
                return c2

            lax.fori_loop(0, G, fma_body, 0, unroll=False)

        prep(jnp.int32(0), jnp.int32(0), 0)
        gather_copy(0).start()

        def pair_body(p, carry):
            for s in (0, 1):
                bt = 2 * p + s
                o = 1 - s
                n_loc = bt // OUT_H
                j7 = bt - n_loc * OUT_H

                n_nxt = (bt + 1) // OUT_H
                j7_nxt = (bt + 1) - n_nxt * OUT_H

                @pl.when(bt < nbatch - 1)
                def _():
                    prep(n_nxt, j7_nxt, o)
                    gather_copy(o).start()

                gather_copy(s).wait()

                @pl.when((j7 == 0) & (n_loc >= 2) & (n_loc % 2 == 0))
                def _():
                    out_copy(n_loc - 2, osem0).wait()

                @pl.when((j7 == 0) & (n_loc >= 2) & (n_loc % 2 == 1))
                def _():
                    out_copy(n_loc - 2, osem1).wait()

                fma(n_loc, j7, s)

                @pl.when((j7 == OUT_H - 1) & (n_loc % 2 == 0))
                def _():
                    out_copy(n_loc, osem0).start()

                @pl.when((j7 == OUT_H - 1) & (n_loc % 2 == 1))
                def _():
                    out_copy(n_loc, osem1).start()
            return carry

        lax.fori_loop(0, nbatch // 2, pair_body, 0, unroll=False)
        out_copy(rois_per_w - 2, osem0).wait()
        out_copy(rois_per_w - 1, osem1).wait()

    return k(feat_rows, roif)


def kernel(features, rois):
    B, C, H, W = features.shape
    N = rois.shape[0]
    feat_rows = (jnp.transpose(features, (0, 2, 3, 1))
                 .astype(jnp.bfloat16).reshape(B * H * W, C))
    offset = 0.5
    cx = rois[:, 1] * SPATIAL_SCALE - offset
    cy = rois[:, 2] * SPATIAL_SCALE - offset
    bw = rois[:, 3] * (SPATIAL_SCALE / OUT_W)
    bh = rois[:, 4] * (SPATIAL_SCALE / OUT_H)
    theta = rois[:, 5]
    base = rois[:, 0].astype(jnp.int32).astype(jnp.float32) * float(H * W)
    z = jnp.zeros_like(cx)
    roif = jnp.stack(
        [cx, cy, bw, bh, jnp.cos(theta), jnp.sin(theta), base,
         z, z, z, z, z, z, z, z, z], axis=1)
    out = _sc_roi_align(feat_rows, roif, H, W, C, N)
    return out.reshape(N, C, OUT_H, OUT_W)
